# Initial kernel scaffold; baseline (speedup 1.0000x reference)
import functools

import jax
import jax.numpy as jnp
from jax import lax
from jax.experimental import pallas as pl
from jax.experimental.pallas import tpu as pltpu
from jax.experimental.pallas import tpu_sc as plsc

N = 10000
E = 320000
D = 128
H = 128
C = 16

NC = 2
NS = 16
NW = NC * NS
EW = E // NW
CH = 80
NCHUNK = EW // CH
NR = N // NS
NP = 10240
NRP = NP // NS

f32 = jnp.float32
i32 = jnp.int32

_mesh = plsc.VectorSubcoreMesh(core_axis_name="c", subcore_axis_name="s")


def _worker_id():
    return lax.axis_index("s") * NC + lax.axis_index("c")


@functools.partial(
    pl.kernel,
    out_type=jax.ShapeDtypeStruct((NC, 2, NP), f32),
    mesh=_mesh,
    scratch_types=[
        pltpu.VMEM((CH,), i32),
        pltpu.VMEM((CH,), i32),
        pltpu.VMEM((CH,), f32),
        pltpu.VMEM((NRP,), f32),
        pltpu.VMEM_SHARED((NP,), f32),
        pltpu.VMEM_SHARED((NP,), f32),
    ],
)
def _sc_degrees(src_hbm, dst_hbm, deg_hbm, idxs_v, idxd_v, ones_v, zbuf_v,
                dego_sh, degi_sh):
    cid = lax.axis_index("c")
    sid = lax.axis_index("s")
    wid = _worker_id()

    @pl.loop(0, CH, step=16)
    def _(j):
        ones_v[pl.ds(j, 16)] = jnp.full((16,), 1.0, f32)

    @pl.loop(0, NRP, step=16)
    def _(j):
        zbuf_v[pl.ds(j, 16)] = jnp.zeros((16,), f32)

    s0 = sid * NRP
    pltpu.sync_copy(zbuf_v, dego_sh.at[pl.ds(s0, NRP)])
    pltpu.sync_copy(zbuf_v, degi_sh.at[pl.ds(s0, NRP)])
    plsc.subcore_barrier()

    base = wid * EW

    @pl.loop(0, NCHUNK)
    def _(ci):
        off = base + ci * CH
        pltpu.sync_copy(src_hbm.at[pl.ds(off, CH)], idxs_v)
        pltpu.sync_copy(dst_hbm.at[pl.ds(off, CH)], idxd_v)
        pltpu.sync_copy(ones_v, dego_sh.at[idxs_v], add=True)
        pltpu.sync_copy(ones_v, degi_sh.at[idxd_v], add=True)

    plsc.subcore_barrier()
    pltpu.sync_copy(dego_sh.at[pl.ds(s0, NRP)], deg_hbm.at[cid, 0, pl.ds(s0, NRP)])
    pltpu.sync_copy(degi_sh.at[pl.ds(s0, NRP)], deg_hbm.at[cid, 1, pl.ds(s0, NRP)])


@functools.partial(
    pl.kernel,
    out_type=jax.ShapeDtypeStruct((NC, N, H), f32),
    mesh=_mesh,
    scratch_types=[
        pltpu.VMEM((CH,), i32),
        pltpu.VMEM((CH,), i32),
        pltpu.VMEM((CH, H), f32),
        pltpu.VMEM((125, H), f32),
        pltpu.VMEM_SHARED((N, H), f32),
        pltpu.SemaphoreType.DMA,
    ],
)
def _sc_agg128(xs_hbm, src_hbm, dst_hbm, agg_hbm, idxs_v, idxd_v, rows_v,
               zrows_v, acc_sh, sem):
    cid = lax.axis_index("c")
    sid = lax.axis_index("s")
    wid = _worker_id()

    @pl.loop(0, 125)
    def _(r):
        @pl.loop(0, H, step=16)
        def _(j):
            zrows_v[r, pl.ds(j, 16)] = jnp.zeros((16,), f32)

    @pl.loop(0, 5)
    def _(k):
        pltpu.sync_copy(zrows_v, acc_sh.at[pl.ds(sid * NR + k * 125, 125), :])

    plsc.subcore_barrier()

    base = wid * EW

    @pl.loop(0, NCHUNK)
    def _(ci):
        off = base + ci * CH
        pltpu.sync_copy(src_hbm.at[pl.ds(off, CH)], idxs_v)
        pltpu.sync_copy(dst_hbm.at[pl.ds(off, CH)], idxd_v)
        pltpu.async_copy(xs_hbm.at[idxs_v], rows_v, sem).wait()
        pltpu.sync_copy(rows_v, acc_sh.at[idxd_v], add=True)

    plsc.subcore_barrier()
    pltpu.sync_copy(acc_sh.at[pl.ds(sid * NR, NR), :],
                    agg_hbm.at[cid, pl.ds(sid * NR, NR), :])


@functools.partial(
    pl.kernel,
    out_type=jax.ShapeDtypeStruct((NC, N, C), f32),
    mesh=_mesh,
    scratch_types=[
        pltpu.VMEM((CH,), i32),
        pltpu.VMEM((CH,), i32),
        pltpu.VMEM((CH, C), f32),
        pltpu.VMEM((NR, C), f32),
        pltpu.VMEM_SHARED((N, C), f32),
        pltpu.VMEM_SHARED((N, C), f32),
        pltpu.SemaphoreType.DMA,
    ],
)
def _sc_agg16(t_hbm, src_hbm, dst_hbm, agg_hbm, idxs_v, idxd_v, rows_v,
              zrows_v, tbl_sh, acc_sh, sem):
    cid = lax.axis_index("c")
    sid = lax.axis_index("s")
    wid = _worker_id()

    @pl.loop(0, NR)
    def _(r):
        zrows_v[r, pl.ds(0, 16)] = jnp.zeros((16,), f32)

    pltpu.sync_copy(zrows_v, acc_sh.at[pl.ds(sid * NR, NR), :])
    pltpu.sync_copy(t_hbm.at[pl.ds(sid * NR, NR), :],
                    tbl_sh.at[pl.ds(sid * NR, NR), :])
    plsc.subcore_barrier()

    base = wid * EW

    @pl.loop(0, NCHUNK)
    def _(ci):
        off = base + ci * CH
        pltpu.sync_copy(src_hbm.at[pl.ds(off, CH)], idxs_v)
        pltpu.sync_copy(dst_hbm.at[pl.ds(off, CH)], idxd_v)
        pltpu.async_copy(tbl_sh.at[idxs_v], rows_v, sem).wait()
        pltpu.sync_copy(rows_v, acc_sh.at[idxd_v], add=True)

    plsc.subcore_barrier()
    pltpu.sync_copy(acc_sh.at[pl.ds(sid * NR, NR), :],
                    agg_hbm.at[cid, pl.ds(sid * NR, NR), :])


_RB = 1000


def _mm1_body(x_ref, w_ref, o_ref):
    o_ref[...] = jnp.dot(x_ref[...], w_ref[...], preferred_element_type=f32)


def _tc_mm1(x, W1):
    return pl.pallas_call(
        _mm1_body,
        grid=(N // _RB,),
        in_specs=[
            pl.BlockSpec((_RB, D), lambda i: (i, 0)),
            pl.BlockSpec((D, H), lambda i: (0, 0)),
        ],
        out_specs=pl.BlockSpec((_RB, H), lambda i: (i, 0)),
        out_shape=jax.ShapeDtypeStruct((N, H), f32),
    )(x, W1)


def _scale_body(y_ref, deg_ref, o_ref):
    do = deg_ref[0, 0, :] + deg_ref[1, 0, :]
    ns = jnp.where(do > 0, lax.rsqrt(do), 0.0)
    o_ref[...] = y_ref[...] * ns[:, None]


def _tc_scale(y1, deg):
    return pl.pallas_call(
        _scale_body,
        grid=(N // _RB,),
        in_specs=[
            pl.BlockSpec((_RB, H), lambda i: (i, 0)),
            pl.BlockSpec((NC, 2, _RB), lambda i: (0, 0, i)),
        ],
        out_specs=pl.BlockSpec((_RB, H), lambda i: (i, 0)),
        out_shape=jax.ShapeDtypeStruct((N, H), f32),
    )(y1, deg)


def _mid_body(agg_ref, deg_ref, b1_ref, w2_ref, o_ref):
    a = agg_ref[0] + agg_ref[1]
    di = deg_ref[0, 1, :] + deg_ref[1, 1, :]
    do = deg_ref[0, 0, :] + deg_ref[1, 0, :]
    nd = jnp.where(di > 0, lax.rsqrt(di), 0.0)
    ns = jnp.where(do > 0, lax.rsqrt(do), 0.0)
    h = jnp.maximum(a * nd[:, None] + b1_ref[...][None, :], 0.0)
    o_ref[...] = jnp.dot(h * ns[:, None], w2_ref[...],
                         preferred_element_type=f32)


def _tc_mid(agg, deg, b1, W2):
    return pl.pallas_call(
        _mid_body,
        grid=(N // _RB,),
        in_specs=[
            pl.BlockSpec((NC, _RB, H), lambda i: (0, i, 0)),
            pl.BlockSpec((NC, 2, _RB), lambda i: (0, 0, i)),
            pl.BlockSpec((H,), lambda i: (0,)),
            pl.BlockSpec((H, C), lambda i: (0, 0)),
        ],
        out_specs=pl.BlockSpec((_RB, C), lambda i: (i, 0)),
        out_shape=jax.ShapeDtypeStruct((N, C), f32),
    )(agg, deg, b1, W2)


def _fin_body(agg_ref, deg_ref, b2_ref, o_ref):
    a = agg_ref[0] + agg_ref[1]
    di = deg_ref[0, 1, :] + deg_ref[1, 1, :]
    nd = jnp.where(di > 0, lax.rsqrt(di), 0.0)
    o_ref[...] = a * nd[:, None] + b2_ref[...][None, :]


def _tc_fin(agg2, deg, b2):
    return pl.pallas_call(
        _fin_body,
        grid=(N // _RB,),
        in_specs=[
            pl.BlockSpec((NC, _RB, C), lambda i: (0, i, 0)),
            pl.BlockSpec((NC, 2, _RB), lambda i: (0, 0, i)),
            pl.BlockSpec((C,), lambda i: (0,)),
        ],
        out_specs=pl.BlockSpec((_RB, C), lambda i: (i, 0)),
        out_shape=jax.ShapeDtypeStruct((N, C), f32),
    )(agg2, deg, b2)


def kernel(x, edge_index, W1, b1, W2, b2):
    src = edge_index[0]
    dst = edge_index[1]
    deg = _sc_degrees(src, dst)
    y1 = _tc_mm1(x, W1)
    xs = _tc_scale(y1, deg)
    agg = _sc_agg128(xs, src, dst)
    t = _tc_mid(agg, deg, b1, W2)
    agg2 = _sc_agg16(t, src, dst)
    return _tc_fin(agg2, deg, b2)


# speedup vs baseline: 4.9112x; 4.9112x over previous
import dataclasses
import functools

import jax
import jax.numpy as jnp
from jax import lax
from jax.experimental import pallas as pl
from jax.experimental.pallas import tpu as pltpu
from jax.experimental.pallas import tpu_sc as plsc

N = 10000
E = 320000
D = 128
H = 128
C = 16

NC = 2
NS = 16
NW = NC * NS
EW = E // NW
CH = 80
NCHUNK = EW // CH
NP = 10240
NRP = NP // NS
NZC = NRP // CH

f32 = jnp.float32
i32 = jnp.int32

_mesh = plsc.VectorSubcoreMesh(core_axis_name="c", subcore_axis_name="s")

_cp = pltpu.CompilerParams()
if "needs_layout_passes" in pltpu.CompilerParams.__dataclass_fields__:
    _cp = dataclasses.replace(_cp, needs_layout_passes=False)


@functools.partial(
    pl.kernel,
    out_type=(
        jax.ShapeDtypeStruct((NC, NS, NP), f32),
        jax.ShapeDtypeStruct((NC, NS, NP), f32),
    ),
    mesh=_mesh,
    scratch_types=[
        pltpu.VMEM((1, CH), i32),
        pltpu.VMEM((1, CH), i32),
        pltpu.VMEM((NP,), f32),
        pltpu.VMEM((NP,), f32),
    ],
    compiler_params=_cp,
)
def _sc_degcnt(src_hbm, dst_hbm, dego_hbm, degi_hbm, si_v, di_v, co_v, ci_v):
    cid = lax.axis_index("c")
    sid = lax.axis_index("s")
    wid = sid * NC + cid

    @pl.loop(0, NP, step=16)
    def _(j):
        co_v[pl.ds(j, 16)] = jnp.zeros((16,), f32)
        ci_v[pl.ds(j, 16)] = jnp.zeros((16,), f32)

    ones = jnp.full((16,), 1.0, f32)
    base = wid * EW

    @pl.loop(0, NCHUNK)
    def _(ck):
        off = base + ck * CH
        pltpu.sync_copy(src_hbm.at[pl.ds(off, CH)], si_v.at[0])
        pltpu.sync_copy(dst_hbm.at[pl.ds(off, CH)], di_v.at[0])

        @pl.loop(0, CH, step=16)
        def _(j):
            plsc.addupdate_scatter(co_v, [si_v[0, pl.ds(j, 16)]], ones)
            plsc.addupdate_scatter(ci_v, [di_v[0, pl.ds(j, 16)]], ones)

    pltpu.sync_copy(co_v, dego_hbm.at[cid, sid, :])
    pltpu.sync_copy(ci_v, degi_hbm.at[cid, sid, :])


@functools.partial(
    pl.kernel,
    out_type=jax.ShapeDtypeStruct((NC, NP, H), f32),
    mesh=_mesh,
    scratch_types=[
        pltpu.VMEM((1, CH), i32),
        pltpu.VMEM((1, CH), i32),
        pltpu.VMEM((CH, H), f32),
        pltpu.VMEM((CH, H), f32),
        pltpu.VMEM((NZC, CH), i32),
        pltpu.VMEM_SHARED((NP, H), f32),
        pltpu.SemaphoreType.DMA,
    ],
)
def _sc_agg128(tbl_hbm, src_hbm, dst_hbm, agg_hbm, si_v, di_v, rows_v,
               zrows_v, own_v, acc_sh, sem):
    cid = lax.axis_index("c")
    sid = lax.axis_index("s")
    wid = sid * NC + cid
    s0 = sid * NRP

    @pl.loop(0, CH)
    def _(r):
        @pl.loop(0, H, step=16)
        def _(j):
            zrows_v[r, pl.ds(j, 16)] = jnp.zeros((16,), f32)

    @pl.loop(0, NZC)
    def _(k):
        @pl.loop(0, CH, step=16)
        def _(j):
            own_v[k, pl.ds(j, 16)] = s0 + k * CH + j + lax.iota(i32, 16)

    @pl.loop(0, NZC)
    def _(k):
        pltpu.sync_copy(zrows_v, acc_sh.at[own_v.at[k]])

    plsc.subcore_barrier()

    base = wid * EW

    @pl.loop(0, NCHUNK)
    def _(ci):
        off = base + ci * CH
        pltpu.sync_copy(src_hbm.at[pl.ds(off, CH)], si_v.at[0])
        pltpu.sync_copy(dst_hbm.at[pl.ds(off, CH)], di_v.at[0])
        pltpu.async_copy(tbl_hbm.at[si_v.at[0]], rows_v, sem).wait()
        pltpu.sync_copy(rows_v, acc_sh.at[di_v.at[0]], add=True)

    plsc.subcore_barrier()

    @pl.loop(0, NZC)
    def _(k):
        pltpu.async_copy(acc_sh.at[own_v.at[k]], zrows_v, sem).wait()
        pltpu.sync_copy(zrows_v, agg_hbm.at[cid, pl.ds(s0 + k * CH, CH), :])


_RB = 1000


def _mm1_body(x_ref, w_ref, o_ref):
    o_ref[...] = jnp.dot(x_ref[...], w_ref[...], preferred_element_type=f32)


def _tc_mm1(x, W1):
    return pl.pallas_call(
        _mm1_body,
        grid=(N // _RB,),
        in_specs=[
            pl.BlockSpec((_RB, D), lambda i: (i, 0)),
            pl.BlockSpec((D, H), lambda i: (0, 0)),
        ],
        out_specs=pl.BlockSpec((_RB, H), lambda i: (i, 0)),
        out_shape=jax.ShapeDtypeStruct((N, H), f32),
    )(x, W1)


def _norms_body(dego_ref, degi_ref, ns_ref, nd_ref):
    do = jnp.sum(dego_ref[...], axis=(0, 1))
    di = jnp.sum(degi_ref[...], axis=(0, 1))
    ns_ref[...] = jnp.where(do > 0, lax.rsqrt(do), 0.0)[:, None]
    nd_ref[...] = jnp.where(di > 0, lax.rsqrt(di), 0.0)[:, None]


def _tc_norms(dego, degi):
    return pl.pallas_call(
        _norms_body,
        grid=(1,),
        in_specs=[
            pl.BlockSpec((NC, NS, NP), lambda i: (0, 0, 0)),
            pl.BlockSpec((NC, NS, NP), lambda i: (0, 0, 0)),
        ],
        out_specs=[
            pl.BlockSpec((NP, 1), lambda i: (0, 0)),
            pl.BlockSpec((NP, 1), lambda i: (0, 0)),
        ],
        out_shape=[
            jax.ShapeDtypeStruct((NP, 1), f32),
            jax.ShapeDtypeStruct((NP, 1), f32),
        ],
    )(dego, degi)


def _scale_body(y_ref, ns_ref, o_ref):
    o_ref[...] = y_ref[...] * ns_ref[...]


def _tc_scale(y1, ns):
    return pl.pallas_call(
        _scale_body,
        grid=(N // _RB,),
        in_specs=[
            pl.BlockSpec((_RB, H), lambda i: (i, 0)),
            pl.BlockSpec((_RB, 1), lambda i: (i, 0)),
        ],
        out_specs=pl.BlockSpec((_RB, H), lambda i: (i, 0)),
        out_shape=jax.ShapeDtypeStruct((N, H), f32),
    )(y1, ns)


def _mid_body(agg_ref, nd_ref, ns_ref, b1_ref, o_ref):
    v = agg_ref[...]
    a = v[0] + v[1]
    h = jnp.maximum(a * nd_ref[...] + b1_ref[...][None, :], 0.0)
    o_ref[...] = h * ns_ref[...]


def _tc_mid(agg, nd, ns, b1):
    return pl.pallas_call(
        _mid_body,
        grid=(N // _RB,),
        in_specs=[
            pl.BlockSpec((NC, _RB, H), lambda i: (0, i, 0)),
            pl.BlockSpec((_RB, 1), lambda i: (i, 0)),
            pl.BlockSpec((_RB, 1), lambda i: (i, 0)),
            pl.BlockSpec((H,), lambda i: (0,)),
        ],
        out_specs=pl.BlockSpec((_RB, H), lambda i: (i, 0)),
        out_shape=jax.ShapeDtypeStruct((N, H), f32),
    )(agg, nd, ns, b1)


def _fin_body(agg_ref, nd_ref, w2_ref, b2_ref, o_ref):
    v = agg_ref[...]
    a = (v[0] + v[1]) * nd_ref[...]
    o_ref[...] = (jnp.dot(a, w2_ref[...], preferred_element_type=f32)
                  + b2_ref[...][None, :])


def _tc_fin(agg2, nd, W2, b2):
    return pl.pallas_call(
        _fin_body,
        grid=(N // _RB,),
        in_specs=[
            pl.BlockSpec((NC, _RB, H), lambda i: (0, i, 0)),
            pl.BlockSpec((_RB, 1), lambda i: (i, 0)),
            pl.BlockSpec((H, C), lambda i: (0, 0)),
            pl.BlockSpec((C,), lambda i: (0,)),
        ],
        out_specs=pl.BlockSpec((_RB, C), lambda i: (i, 0)),
        out_shape=jax.ShapeDtypeStruct((N, C), f32),
    )(agg2, nd, W2, b2)


def kernel(x, edge_index, W1, b1, W2, b2):
    src = edge_index[0]
    dst = edge_index[1]
    dego, degi = _sc_degcnt(src, dst)
    y1 = _tc_mm1(x, W1)
    ns, nd = _tc_norms(dego, degi)
    xs = _tc_scale(y1, ns[:N])
    agg = _sc_agg128(xs, src, dst)
    hs = _tc_mid(agg, nd[:N], ns[:N], b1)
    agg2 = _sc_agg128(hs, src, dst)
    return _tc_fin(agg2, nd[:N], W2, b2)


# speedup vs baseline: 7.1276x; 1.4513x over previous
import dataclasses
import functools

import jax
import jax.numpy as jnp
from jax import lax
from jax.experimental import pallas as pl
from jax.experimental.pallas import tpu as pltpu
from jax.experimental.pallas import tpu_sc as plsc

N = 10000
E = 320000
D = 128
H = 128
C = 16

NC = 2
NS = 16
NW = NC * NS
EW = E // NW
CH = 80
NCHUNK = EW // CH
NP = 10240
NRP = NP // NS
NZC = NRP // CH

f32 = jnp.float32
i32 = jnp.int32

_mesh = plsc.VectorSubcoreMesh(core_axis_name="c", subcore_axis_name="s")

_cp = pltpu.CompilerParams()
if "needs_layout_passes" in pltpu.CompilerParams.__dataclass_fields__:
    _cp = dataclasses.replace(_cp, needs_layout_passes=False)


@functools.partial(
    pl.kernel,
    out_type=(
        jax.ShapeDtypeStruct((NC, NS, NP), f32),
        jax.ShapeDtypeStruct((NC, NS, NP), f32),
    ),
    mesh=_mesh,
    scratch_types=[
        pltpu.VMEM((1, CH), i32),
        pltpu.VMEM((1, CH), i32),
        pltpu.VMEM((NP,), f32),
        pltpu.VMEM((NP,), f32),
    ],
    compiler_params=_cp,
)
def _sc_degcnt(src_hbm, dst_hbm, dego_hbm, degi_hbm, si_v, di_v, co_v, ci_v):
    cid = lax.axis_index("c")
    sid = lax.axis_index("s")
    wid = sid * NC + cid

    @pl.loop(0, NP, step=16)
    def _(j):
        co_v[pl.ds(j, 16)] = jnp.zeros((16,), f32)
        ci_v[pl.ds(j, 16)] = jnp.zeros((16,), f32)

    ones = jnp.full((16,), 1.0, f32)
    base = wid * EW

    @pl.loop(0, NCHUNK)
    def _(ck):
        off = base + ck * CH
        pltpu.sync_copy(src_hbm.at[pl.ds(off, CH)], si_v.at[0])
        pltpu.sync_copy(dst_hbm.at[pl.ds(off, CH)], di_v.at[0])

        @pl.loop(0, CH, step=16)
        def _(j):
            plsc.addupdate_scatter(co_v, [si_v[0, pl.ds(j, 16)]], ones)
            plsc.addupdate_scatter(ci_v, [di_v[0, pl.ds(j, 16)]], ones)

    pltpu.sync_copy(co_v, dego_hbm.at[cid, sid, :])
    pltpu.sync_copy(ci_v, degi_hbm.at[cid, sid, :])


@functools.partial(
    pl.kernel,
    out_type=jax.ShapeDtypeStruct((NC, NP, H), f32),
    mesh=_mesh,
    scratch_types=[
        pltpu.VMEM((1, CH), i32),
        pltpu.VMEM((1, CH), i32),
        pltpu.VMEM((1, CH), i32),
        pltpu.VMEM((1, CH), i32),
        pltpu.VMEM((CH, H), f32),
        pltpu.VMEM((CH, H), f32),
        pltpu.VMEM((CH, H), f32),
        pltpu.VMEM((NZC, CH), i32),
        pltpu.VMEM_SHARED((NP, H), f32),
        pltpu.SemaphoreType.DMA,
        pltpu.SemaphoreType.DMA,
    ],
)
def _sc_agg128(tbl_hbm, src_hbm, dst_hbm, agg_hbm, si0, di0, si1, di1,
               rows0, rows1, zrows_v, own_v, acc_sh, sem0, sem1):
    cid = lax.axis_index("c")
    sid = lax.axis_index("s")
    wid = sid * NC + cid
    s0 = sid * NRP

    @pl.loop(0, CH)
    def _(r):
        @pl.loop(0, H, step=16)
        def _(j):
            zrows_v[r, pl.ds(j, 16)] = jnp.zeros((16,), f32)

    @pl.loop(0, NZC)
    def _(k):
        @pl.loop(0, CH, step=16)
        def _(j):
            own_v[k, pl.ds(j, 16)] = s0 + k * CH + j + lax.iota(i32, 16)

    @pl.loop(0, NZC)
    def _(k):
        pltpu.sync_copy(zrows_v, acc_sh.at[own_v.at[k]])

    plsc.subcore_barrier()

    base = wid * EW

    pltpu.sync_copy(src_hbm.at[pl.ds(base, CH)], si0.at[0])
    pltpu.sync_copy(dst_hbm.at[pl.ds(base, CH)], di0.at[0])
    pltpu.async_copy(tbl_hbm.at[si0.at[0]], rows0, sem0)

    @pl.loop(0, NCHUNK - 1, step=2)
    def _(ci):
        off1 = base + (ci + 1) * CH
        pltpu.sync_copy(src_hbm.at[pl.ds(off1, CH)], si1.at[0])
        pltpu.sync_copy(dst_hbm.at[pl.ds(off1, CH)], di1.at[0])
        pltpu.async_copy(tbl_hbm.at[si1.at[0]], rows1, sem1)

        pltpu.make_async_copy(tbl_hbm.at[si0.at[0]], rows0, sem0).wait()
        pltpu.sync_copy(rows0, acc_sh.at[di0.at[0]], add=True)

        off2 = base + (ci + 2) * CH
        pltpu.sync_copy(src_hbm.at[pl.ds(off2, CH)], si0.at[0])
        pltpu.sync_copy(dst_hbm.at[pl.ds(off2, CH)], di0.at[0])
        pltpu.async_copy(tbl_hbm.at[si0.at[0]], rows0, sem0)

        pltpu.make_async_copy(tbl_hbm.at[si1.at[0]], rows1, sem1).wait()
        pltpu.sync_copy(rows1, acc_sh.at[di1.at[0]], add=True)

    pltpu.make_async_copy(tbl_hbm.at[si0.at[0]], rows0, sem0).wait()
    pltpu.sync_copy(rows0, acc_sh.at[di0.at[0]], add=True)

    plsc.subcore_barrier()

    @pl.loop(0, NZC)
    def _(k):
        pltpu.async_copy(acc_sh.at[own_v.at[k]], zrows_v, sem0).wait()
        pltpu.sync_copy(zrows_v, agg_hbm.at[cid, pl.ds(s0 + k * CH, CH), :])


_RB = 1000


def _mm1_body(x_ref, w_ref, o_ref):
    o_ref[...] = jnp.dot(x_ref[...], w_ref[...], preferred_element_type=f32)


def _tc_mm1(x, W1):
    return pl.pallas_call(
        _mm1_body,
        grid=(N // _RB,),
        in_specs=[
            pl.BlockSpec((_RB, D), lambda i: (i, 0)),
            pl.BlockSpec((D, H), lambda i: (0, 0)),
        ],
        out_specs=pl.BlockSpec((_RB, H), lambda i: (i, 0)),
        out_shape=jax.ShapeDtypeStruct((N, H), f32),
    )(x, W1)


def _norms_body(dego_ref, degi_ref, ns_ref, nd_ref):
    do = jnp.sum(dego_ref[...], axis=(0, 1))
    di = jnp.sum(degi_ref[...], axis=(0, 1))
    ns_ref[...] = jnp.where(do > 0, lax.rsqrt(do), 0.0)[:, None]
    nd_ref[...] = jnp.where(di > 0, lax.rsqrt(di), 0.0)[:, None]


def _tc_norms(dego, degi):
    return pl.pallas_call(
        _norms_body,
        grid=(1,),
        in_specs=[
            pl.BlockSpec((NC, NS, NP), lambda i: (0, 0, 0)),
            pl.BlockSpec((NC, NS, NP), lambda i: (0, 0, 0)),
        ],
        out_specs=[
            pl.BlockSpec((NP, 1), lambda i: (0, 0)),
            pl.BlockSpec((NP, 1), lambda i: (0, 0)),
        ],
        out_shape=[
            jax.ShapeDtypeStruct((NP, 1), f32),
            jax.ShapeDtypeStruct((NP, 1), f32),
        ],
    )(dego, degi)


def _scale_body(y_ref, ns_ref, o_ref):
    o_ref[...] = y_ref[...] * ns_ref[...]


def _tc_scale(y1, ns):
    return pl.pallas_call(
        _scale_body,
        grid=(N // _RB,),
        in_specs=[
            pl.BlockSpec((_RB, H), lambda i: (i, 0)),
            pl.BlockSpec((_RB, 1), lambda i: (i, 0)),
        ],
        out_specs=pl.BlockSpec((_RB, H), lambda i: (i, 0)),
        out_shape=jax.ShapeDtypeStruct((N, H), f32),
    )(y1, ns)


def _mid_body(agg_ref, nd_ref, ns_ref, b1_ref, o_ref):
    v = agg_ref[...]
    a = v[0] + v[1]
    h = jnp.maximum(a * nd_ref[...] + b1_ref[...][None, :], 0.0)
    o_ref[...] = h * ns_ref[...]


def _tc_mid(agg, nd, ns, b1):
    return pl.pallas_call(
        _mid_body,
        grid=(N // _RB,),
        in_specs=[
            pl.BlockSpec((NC, _RB, H), lambda i: (0, i, 0)),
            pl.BlockSpec((_RB, 1), lambda i: (i, 0)),
            pl.BlockSpec((_RB, 1), lambda i: (i, 0)),
            pl.BlockSpec((H,), lambda i: (0,)),
        ],
        out_specs=pl.BlockSpec((_RB, H), lambda i: (i, 0)),
        out_shape=jax.ShapeDtypeStruct((N, H), f32),
    )(agg, nd, ns, b1)


def _fin_body(agg_ref, nd_ref, w2_ref, b2_ref, o_ref):
    v = agg_ref[...]
    a = (v[0] + v[1]) * nd_ref[...]
    o_ref[...] = (jnp.dot(a, w2_ref[...], preferred_element_type=f32)
                  + b2_ref[...][None, :])


def _tc_fin(agg2, nd, W2, b2):
    return pl.pallas_call(
        _fin_body,
        grid=(N // _RB,),
        in_specs=[
            pl.BlockSpec((NC, _RB, H), lambda i: (0, i, 0)),
            pl.BlockSpec((_RB, 1), lambda i: (i, 0)),
            pl.BlockSpec((H, C), lambda i: (0, 0)),
            pl.BlockSpec((C,), lambda i: (0,)),
        ],
        out_specs=pl.BlockSpec((_RB, C), lambda i: (i, 0)),
        out_shape=jax.ShapeDtypeStruct((N, C), f32),
    )(agg2, nd, W2, b2)


def kernel(x, edge_index, W1, b1, W2, b2):
    src = edge_index[0]
    dst = edge_index[1]
    dego, degi = _sc_degcnt(src, dst)
    y1 = _tc_mm1(x, W1)
    ns, nd = _tc_norms(dego, degi)
    xs = _tc_scale(y1, ns[:N])
    agg = _sc_agg128(xs, src, dst)
    hs = _tc_mid(agg, nd[:N], ns[:N], b1)
    agg2 = _sc_agg128(hs, src, dst)
    return _tc_fin(agg2, nd[:N], W2, b2)
